# baseline (speedup 1.0000x reference)
import jax, jax.numpy as jnp
from jax.experimental import pallas as pl
from jax.experimental.pallas import tpu as pltpu

_N = 16 * 3 * 512 * 512

def _body(x_ref, o_ref, sem):
    c = pltpu.make_async_copy(x_ref, o_ref, sem)
    c.start(); c.wait()

def kernel(x):
    flat = x.reshape(_N)
    out = pl.pallas_call(
        _body,
        in_specs=[pl.BlockSpec(memory_space=pltpu.MemorySpace.HBM)],
        out_specs=pl.BlockSpec(memory_space=pltpu.MemorySpace.HBM),
        out_shape=jax.ShapeDtypeStruct((_N,), jnp.float32),
        scratch_shapes=[pltpu.SemaphoreType.DMA],
    )(flat)
    return out.reshape(x.shape)


# speedup vs baseline: 11.8187x; 11.8187x over previous
import jax
import jax.numpy as jnp
from jax.experimental import pallas as pl
from jax.experimental.pallas import tpu as pltpu

_CHUNKS = 24


def _dma_copy(x_ref, o_ref, scratch, in_sems, out_sems):
    rows = x_ref.shape[0]
    ch = rows // _CHUNKS

    def in_copy(i):
        return pltpu.make_async_copy(
            x_ref.at[pl.ds(i * ch, ch)], scratch.at[i], in_sems.at[i])

    def out_copy(i):
        return pltpu.make_async_copy(
            scratch.at[i], o_ref.at[pl.ds(i * ch, ch)], out_sems.at[i])

    for i in range(_CHUNKS):
        in_copy(i).start()
    for i in range(_CHUNKS):
        in_copy(i).wait()
        out_copy(i).start()
    for i in range(_CHUNKS):
        out_copy(i).wait()


def kernel(x):
    b, c, h, w = x.shape
    flat = x.reshape(b * c * h // 2, w * 2)
    rows, cols = flat.shape
    ch = rows // _CHUNKS
    out = pl.pallas_call(
        _dma_copy,
        in_specs=[pl.BlockSpec(memory_space=pltpu.MemorySpace.HBM)],
        out_specs=pl.BlockSpec(memory_space=pltpu.MemorySpace.HBM),
        out_shape=jax.ShapeDtypeStruct(flat.shape, flat.dtype),
        scratch_shapes=[
            pltpu.VMEM((_CHUNKS, ch, cols), jnp.float32),
            pltpu.SemaphoreType.DMA((_CHUNKS,)),
            pltpu.SemaphoreType.DMA((_CHUNKS,)),
        ],
        compiler_params=pltpu.CompilerParams(
            vmem_limit_bytes=56 * 1024 * 1024,
        ),
    )(flat)
    return out.reshape(x.shape)
